# Initial kernel scaffold; baseline (speedup 1.0000x reference)
import jax
import jax.numpy as jnp
from jax.experimental import pallas as pl


def kernel(task_features, node_features, neighbor_relations, neighbor_features, task_emb, node_emb, rel_emb, neigh_emb, W_task_mul, b_task_mul, W_node_mul, b_node_mul, W_rel_mul, b_rel_mul, W_neigh_mul, b_neigh_mul, W_task_sum, b_task_sum, W_node_sum, b_node_sum, W_rel_sum, b_rel_sum, W_neigh_sum, b_neigh_sum, W_out, b_out):
    raise NotImplementedError("write your pallas kernel here")


# speedup vs baseline: 1.0888x; 1.0888x over previous
import functools

import jax
import jax.numpy as jnp
from jax import lax
from jax.experimental import pallas as pl
from jax.experimental.pallas import tpu as pltpu
from jax.experimental.pallas import tpu_sc as plsc

N = 320000
NF = 4
TF = 8
D = 128
H = 128
F2 = 2 * H
VN = 10000
VR = 16
VT = 1000

NC = 2
NS = 16
NW = NC * NS
PER_W = N // NW
CH = 16
NCH = PER_W // CH
ROWS = NF * CH


def _project_body(ne_ref, ge_ref, wn_ref, wg_ref, bn_ref, bg_ref, pn_ref, pg_ref):
    pn_ref[...] = (jnp.dot(ne_ref[...], wn_ref[...],
                           preferred_element_type=jnp.float32) + bn_ref[...]) * 0.25
    pg_ref[...] = (jnp.dot(ge_ref[...], wg_ref[...],
                           preferred_element_type=jnp.float32) + bg_ref[...]) * 0.25


def _ab_body(te_ref, tf_ref, re_ref, wtm_ref, wts_ref, wrm_ref, wrs_ref,
             btm_ref, bts_ref, brm_ref, brs_ref, ab_ref):
    ids = lax.broadcasted_iota(jnp.int32, (TF, VT), 1)
    onehot = (ids == tf_ref[...]).astype(jnp.float32)
    cnt = jnp.sum(onehot, axis=0, keepdims=True) * (1.0 / TF)
    te = jnp.dot(cnt, te_ref[...], preferred_element_type=jnp.float32)
    mt = jnp.dot(te, wtm_ref[...], preferred_element_type=jnp.float32) + btm_ref[...]
    st = jnp.dot(te, wts_ref[...], preferred_element_type=jnp.float32) + bts_ref[...]
    mr = jnp.dot(re_ref[...], wrm_ref[...], preferred_element_type=jnp.float32) + brm_ref[...]
    sr = jnp.dot(re_ref[...], wrs_ref[...], preferred_element_type=jnp.float32) + brs_ref[...]
    ab_ref[:, :H] = mt * mr
    ab_ref[:, H:] = st + sr


_MESH = plsc.VectorSubcoreMesh(core_axis_name="c", subcore_axis_name="s")


@functools.partial(
    pl.kernel,
    mesh=_MESH,
    out_type=jax.ShapeDtypeStruct((N,), jnp.float32),
    compiler_params=pltpu.CompilerParams(needs_layout_passes=False),
    scratch_types=[
        pltpu.VMEM((ROWS,), jnp.int32),
        pltpu.VMEM((ROWS,), jnp.int32),
        pltpu.VMEM((ROWS,), jnp.int32),
        pltpu.VMEM((ROWS,), jnp.int32),
        pltpu.VMEM((CH,), jnp.int32),
        pltpu.VMEM((CH,), jnp.int32),
        pltpu.VMEM((ROWS, F2), jnp.float32),
        pltpu.VMEM((ROWS, F2), jnp.float32),
        pltpu.VMEM((ROWS, F2), jnp.float32),
        pltpu.VMEM((ROWS, F2), jnp.float32),
        pltpu.VMEM((VR, F2), jnp.float32),
        pltpu.VMEM((F2,), jnp.float32),
        pltpu.VMEM((PER_W,), jnp.float32),
        pltpu.SemaphoreType.DMA,
        pltpu.SemaphoreType.DMA,
        pltpu.SemaphoreType.DMA,
        pltpu.SemaphoreType.DMA,
        pltpu.SemaphoreType.DMA,
        pltpu.SemaphoreType.DMA,
        pltpu.SemaphoreType.DMA,
        pltpu.SemaphoreType.DMA,
        pltpu.SemaphoreType.DMA,
        pltpu.SemaphoreType.DMA,
    ],
)
def _sc_kernel(nf_hbm, gf_hbm, rel_hbm, pn_hbm, pg_hbm, ab_hbm, wb_hbm, out_hbm,
               idx_n0, idx_n1, idx_g0, idx_g1, idx_r0, idx_r1,
               rows_n0, rows_n1, rows_g0, rows_g1, ab_v, wb_v, out_v,
               s_n0, s_n1, s_g0, s_g1, s_r0, s_r1, s_rn0, s_rn1, s_rg0, s_rg1):
    wid = lax.axis_index("s") * NC + lax.axis_index("c")
    base = wid * PER_W

    pltpu.sync_copy(ab_hbm, ab_v)
    pltpu.sync_copy(wb_hbm, wb_v)

    bufs = (
        (idx_n0, idx_g0, idx_r0, rows_n0, rows_g0, s_n0, s_g0, s_r0, s_rn0, s_rg0),
        (idx_n1, idx_g1, idx_r1, rows_n1, rows_g1, s_n1, s_g1, s_r1, s_rn1, s_rg1),
    )

    def idx_copies(c, b):
        cb = base + c * CH
        return (
            pltpu.make_async_copy(nf_hbm.at[pl.ds(cb * NF, ROWS)], b[0], b[5]),
            pltpu.make_async_copy(gf_hbm.at[pl.ds(cb * NF, ROWS)], b[1], b[6]),
            pltpu.make_async_copy(rel_hbm.at[pl.ds(cb, CH)], b[2], b[7]),
        )

    def row_copies(b):
        return (
            pltpu.make_async_copy(pn_hbm.at[b[0]], b[3], b[8]),
            pltpu.make_async_copy(pg_hbm.at[b[1]], b[4], b[9]),
        )

    iota = lax.iota(jnp.int32, CH)
    row_k = [iota * NF + k for k in range(NF)]
    bvec = None

    def compute(c, b):
        rv = b[2][...]
        rows_n = b[3]
        rows_g = b[4]

        def hbody(h, acc):
            hv = jnp.full((CH,), h, jnp.int32)
            hv2 = hv + H
            mn = (plsc.load_gather(rows_n, [row_k[0], hv])
                  + plsc.load_gather(rows_n, [row_k[1], hv])
                  + plsc.load_gather(rows_n, [row_k[2], hv])
                  + plsc.load_gather(rows_n, [row_k[3], hv]))
            sn = (plsc.load_gather(rows_n, [row_k[0], hv2])
                  + plsc.load_gather(rows_n, [row_k[1], hv2])
                  + plsc.load_gather(rows_n, [row_k[2], hv2])
                  + plsc.load_gather(rows_n, [row_k[3], hv2]))
            mg = (plsc.load_gather(rows_g, [row_k[0], hv])
                  + plsc.load_gather(rows_g, [row_k[1], hv])
                  + plsc.load_gather(rows_g, [row_k[2], hv])
                  + plsc.load_gather(rows_g, [row_k[3], hv]))
            sg = (plsc.load_gather(rows_g, [row_k[0], hv2])
                  + plsc.load_gather(rows_g, [row_k[1], hv2])
                  + plsc.load_gather(rows_g, [row_k[2], hv2])
                  + plsc.load_gather(rows_g, [row_k[3], hv2]))
            a = plsc.load_gather(ab_v, [rv, hv])
            bb = plsc.load_gather(ab_v, [rv, hv2])
            w = plsc.load_gather(wb_v, [hv])
            hid = jnp.maximum(a * mn * mg + bb + sn + sg, 0.0)
            return acc + hid * w

        acc = lax.fori_loop(0, H, hbody, jnp.zeros((CH,), jnp.float32))
        out_v[pl.ds(c * CH, CH)] = acc + bvec

    d0 = idx_copies(0, bufs[0])
    for x in d0:
        x.start()
    d1 = idx_copies(1, bufs[1])
    for x in d1:
        x.start()
    for x in d0:
        x.wait()
    for x in row_copies(bufs[0]):
        x.start()

    bvec = plsc.load_gather(wb_v, [jnp.full((CH,), H, jnp.int32)])

    def do_chunk(c, cur, nxt):
        for x in row_copies(cur):
            x.wait()

        @pl.when(c + 1 < NCH)
        def _():
            for x in idx_copies(c + 1, nxt):
                x.wait()
            for x in row_copies(nxt):
                x.start()

        @pl.when(c + 2 < NCH)
        def _():
            for x in idx_copies(c + 2, cur):
                x.start()

        compute(c, cur)

    def loop_body(g, carry):
        c0 = 2 * g
        do_chunk(c0, bufs[0], bufs[1])

        @pl.when(c0 + 1 < NCH)
        def _():
            do_chunk(c0 + 1, bufs[1], bufs[0])

        return carry

    lax.fori_loop(0, (NCH + 1) // 2, loop_body, 0)
    pltpu.sync_copy(out_v, out_hbm.at[pl.ds(base, PER_W)])


def kernel(task_features, node_features, neighbor_relations, neighbor_features,
           task_emb, node_emb, rel_emb, neigh_emb,
           W_task_mul, b_task_mul, W_node_mul, b_node_mul,
           W_rel_mul, b_rel_mul, W_neigh_mul, b_neigh_mul,
           W_task_sum, b_task_sum, W_node_sum, b_node_sum,
           W_rel_sum, b_rel_sum, W_neigh_sum, b_neigh_sum,
           W_out, b_out):
    nf = node_features.astype(jnp.int32).reshape(-1)
    gf = neighbor_features.astype(jnp.int32).reshape(-1)
    rel = neighbor_relations.astype(jnp.int32)
    tf2 = task_features.astype(jnp.int32).reshape(TF, 1)

    wn = jnp.concatenate([W_node_mul, W_node_sum], axis=1)
    wg = jnp.concatenate([W_neigh_mul, W_neigh_sum], axis=1)
    bn = jnp.concatenate([b_node_mul, b_node_sum]).reshape(1, F2)
    bg = jnp.concatenate([b_neigh_mul, b_neigh_sum]).reshape(1, F2)

    blk = VN // 5
    pn, pg = pl.pallas_call(
        _project_body,
        grid=(5,),
        in_specs=[
            pl.BlockSpec((blk, D), lambda i: (i, 0)),
            pl.BlockSpec((blk, D), lambda i: (i, 0)),
            pl.BlockSpec((D, F2), lambda i: (0, 0)),
            pl.BlockSpec((D, F2), lambda i: (0, 0)),
            pl.BlockSpec((1, F2), lambda i: (0, 0)),
            pl.BlockSpec((1, F2), lambda i: (0, 0)),
        ],
        out_specs=[
            pl.BlockSpec((blk, F2), lambda i: (i, 0)),
            pl.BlockSpec((blk, F2), lambda i: (i, 0)),
        ],
        out_shape=[
            jax.ShapeDtypeStruct((VN, F2), jnp.float32),
            jax.ShapeDtypeStruct((VN, F2), jnp.float32),
        ],
    )(node_emb, neigh_emb, wn, wg, bn, bg)

    ab = pl.pallas_call(
        _ab_body,
        out_shape=jax.ShapeDtypeStruct((VR, F2), jnp.float32),
    )(task_emb, tf2, rel_emb, W_task_mul, W_task_sum, W_rel_mul, W_rel_sum,
      b_task_mul.reshape(1, H), b_task_sum.reshape(1, H),
      b_rel_mul.reshape(1, H), b_rel_sum.reshape(1, H))

    wb = jnp.zeros((F2,), jnp.float32).at[:H].set(W_out[:, 0]).at[H].set(b_out[0])

    return _sc_kernel(nf, gf, rel, pn, pg, ab, wb)
